# baseline (speedup 1.0000x reference)
import functools

import numpy as np
import jax
import jax.numpy as jnp
from jax import lax
from jax.experimental import pallas as pl
from jax.experimental.pallas import tpu as pltpu
from jax.experimental.pallas import tpu_sc as plsc

_G = 58
_M = _G * _G
_OFF = 64
_S = 3488
_V = 512


def _conv_body(in_ref, w_ref, b_ref, o_ref, *, act, cin):
    taps = [in_ref[pl.ds(_OFF + (dh - 1) * _G + (dw - 1), _M), pl.ds(0, cin)]
            for dh in range(3) for dw in range(3)]
    p = jnp.concatenate(taps, axis=1)
    acc = jnp.dot(p, w_ref[...], preferred_element_type=jnp.float32)
    acc = acc + b_ref[...]
    if act == "relu":
        acc = jnp.maximum(acc, 0.0)
    elif act == "sigmoid":
        acc = jax.nn.sigmoid(acc)
    q = jax.lax.broadcasted_iota(jnp.int32, (_M, 1), 0)
    r, c = q // _G, q % _G
    valid = (r >= 1) & (r <= 56) & (c >= 1) & (c <= 56)
    acc = jnp.where(valid, acc, 0.0)
    o_ref[pl.ds(0, _OFF), :] = jnp.zeros((_OFF, acc.shape[1]), jnp.float32)
    o_ref[pl.ds(_OFF, _M), :] = acc
    o_ref[pl.ds(_S - 64, 64), :] = jnp.zeros((64, acc.shape[1]), jnp.float32)


def _conv(x, w, b, act=None):
    bm, cin = x.shape
    nb = bm // _S
    k, n = w.shape
    return pl.pallas_call(
        functools.partial(_conv_body, act=act, cin=k // 9),
        grid=(nb,),
        in_specs=[
            pl.BlockSpec((_S, cin), lambda i: (i, 0)),
            pl.BlockSpec((k, n), lambda i: (0, 0)),
            pl.BlockSpec((1, n), lambda i: (0, 0)),
        ],
        out_specs=pl.BlockSpec((_S, n), lambda i: (i, 0)),
        out_shape=jax.ShapeDtypeStruct((bm, n), jnp.float32),
    )(x, w, b.reshape(1, n))


def _vq_body(z_ref, c_ref, idx_ref, zq_ref, loss_ref):
    z = z_ref[...]
    c = c_ref[...]
    s = jax.lax.dot_general(z, c, (((1,), (1,)), ((), ())),
                            preferred_element_type=jnp.float32)
    z2 = jnp.sum(z * z, axis=1, keepdims=True)
    c2 = jnp.sum(c * c, axis=1)[None, :]
    dists = z2 + c2 - 2.0 * s
    m = jnp.min(dists, axis=1, keepdims=True)
    iota = jax.lax.broadcasted_iota(jnp.int32, dists.shape, 1)
    idx = jnp.min(jnp.where(dists <= m, iota, _V), axis=1)

    q = jax.lax.broadcasted_iota(jnp.int32, (_S, 1), 0) - _OFF
    r, cc = q // _G, q % _G
    valid = (q >= 0) & (q < _M) & (r >= 1) & (r <= 56) & (cc >= 1) & (cc <= 56)
    idx_ref[0, 0, :] = jnp.where(valid[:, 0], idx, _V)
    oh = (iota == idx[:, None]).astype(jnp.float32)
    zq = jax.lax.dot_general(oh, c, (((1,), (0,)), ((), ())),
                             preferred_element_type=jnp.float32)
    zq_ref[...] = jnp.where(valid, zq, 0.0)
    part = jnp.sum(jnp.where(valid, m, 0.0)).reshape(1, 1)

    @pl.when(pl.program_id(0) == 0)
    def _():
        loss_ref[...] = jnp.zeros((1, 1), jnp.float32)

    loss_ref[...] += part


def _vq(z, codebook):
    bm, n = z.shape
    nb = bm // _S
    idx3, zq, loss = pl.pallas_call(
        _vq_body,
        grid=(nb,),
        in_specs=[
            pl.BlockSpec((_S, n), lambda i: (i, 0)),
            pl.BlockSpec((_V, n), lambda i: (0, 0)),
        ],
        out_specs=[
            pl.BlockSpec((1, 1, _S), lambda i: (i, 0, 0)),
            pl.BlockSpec((_S, n), lambda i: (i, 0)),
            pl.BlockSpec((1, 1), lambda i: (0, 0)),
        ],
        out_shape=[
            jax.ShapeDtypeStruct((nb, 1, _S), jnp.int32),
            jax.ShapeDtypeStruct((bm, n), jnp.float32),
            jax.ShapeDtypeStruct((1, 1), jnp.float32),
        ],
    )(z, codebook)
    return idx3, zq, loss[0, 0]


def _sc_gather(table, idx):
    nrows = idx.shape[0]
    d = table.shape[1]
    info = plsc.get_sparse_core_info()
    nw = info.num_cores * info.num_subcores
    b_per_w = nrows // nw
    nch = 4
    bc = b_per_w // nch
    mesh = plsc.VectorSubcoreMesh(core_axis_name="c", subcore_axis_name="s")

    @functools.partial(
        pl.kernel, mesh=mesh,
        out_type=jax.ShapeDtypeStruct((nrows, d), jnp.float32),
        scratch_types=[
            pltpu.VMEM((bc,), jnp.int32),
            pltpu.VMEM((bc, d), jnp.float32),
            pltpu.SemaphoreType.DMA,
        ],
    )
    def k(table_hbm, idx_hbm, out_hbm, idx_v, rows_v, sem):
        wid = lax.axis_index("s") * info.num_cores + lax.axis_index("c")
        base = wid * b_per_w
        for ch in range(nch):
            off = base + ch * bc
            pltpu.sync_copy(idx_hbm.at[pl.ds(off, bc)], idx_v)
            pltpu.async_copy(table_hbm.at[idx_v], rows_v, sem).wait()
            pltpu.sync_copy(rows_v, out_hbm.at[pl.ds(off, bc)])

    return k(table, idx)


def _fold4(v):
    return v // 4, v % 4


def _np_map_enc1():
    widx = np.full((3, 3, 4, 4, 2, 2, 32), -1, np.int64)
    for fh in range(2):
        for th in range(4):
            dh, eh = _fold4(2 * fh + th - 1)
            for fw in range(2):
                for tw in range(4):
                    dw, ew = _fold4(2 * fw + tw - 1)
                    for o in range(32):
                        widx[dh + 1, dw + 1, eh, ew, fh, fw, o] = o * 16 + th * 4 + tw
    return widx.reshape(144, 128)


def _np_map_enc2():
    widx = np.full((3, 3, 2, 2, 32, 64), -1, np.int64)
    for th in range(4):
        dh, fh = (th - 1) // 2, (th - 1) % 2
        for tw in range(4):
            dw, fw = (tw - 1) // 2, (tw - 1) % 2
            for i in range(32):
                for o in range(64):
                    widx[dh + 1, dw + 1, fh, fw, i, o] = o * 512 + i * 16 + th * 4 + tw
    return widx.reshape(1152, 64)


def _np_map_s1(o_ch, i_ch, flipped):
    widx = np.full((3, 3, i_ch, o_ch), -1, np.int64)
    for a in range(3):
        for b in range(3):
            for i in range(i_ch):
                for o in range(o_ch):
                    if flipped:
                        widx[a, b, i, o] = i * (o_ch * 9) + o * 9 + (2 - a) * 3 + (2 - b)
                    else:
                        widx[a, b, i, o] = o * (i_ch * 9) + i * 9 + a * 3 + b
    return widx.reshape(9 * i_ch, o_ch)


_PHASE_TAPS = {0: ((0, 0), (1, 2)), 1: ((1, 1), (2, 3))}


def _np_map_dec2():
    widx = np.full((3, 3, 64, 2, 2, 32), -1, np.int64)
    for r in (0, 1):
        for s in (0, 1):
            for ph, dh in _PHASE_TAPS[r]:
                for pw, dw in _PHASE_TAPS[s]:
                    for i in range(64):
                        for o in range(32):
                            widx[ph, pw, i, r, s, o] = (
                                i * 512 + o * 16 + (3 - dh) * 4 + (3 - dw))
    return widx.reshape(576, 128)


def _np_map_dec3():
    widx = np.full((3, 3, 2, 2, 32, 4, 4), -1, np.int64)
    for fh in range(2):
        for th in range(4):
            dh, eh = _fold4(2 * fh - th + 2)
            for fw in range(2):
                for tw in range(4):
                    dw, ew = _fold4(2 * fw - tw + 2)
                    for c in range(32):
                        widx[1 - dh, 1 - dw, fh, fw, c, eh, ew] = (
                            c * 16 + (3 - th) * 4 + (3 - tw))
    return widx.reshape(1152, 16)


def _const_map(widx):
    mask = (widx >= 0).astype(np.float32)
    return np.maximum(widx, 0).astype(np.int32), mask


_MAP_ENC1 = _const_map(_np_map_enc1())
_MAP_ENC2 = _const_map(_np_map_enc2())
_MAP_ENC3 = _const_map(_np_map_s1(64, 64, False))
_MAP_DEC1 = _const_map(_np_map_s1(64, 64, True))
_MAP_DEC2 = _const_map(_np_map_dec2())
_MAP_DEC3 = _const_map(_np_map_dec3())


def _remap(w, m):
    gidx, mask = m
    return w.reshape(-1)[gidx] * mask


def _to_grid(x):
    b, _, _, c = x.shape
    xp = jnp.pad(x, ((0, 0), (1, 1), (1, 1), (0, 0))).reshape(b, _M, c)
    xp = jnp.pad(xp, ((0, 0), (_OFF, _S - _OFF - _M), (0, 0)))
    return xp.reshape(b * _S, c)


def _from_grid(x, b):
    c = x.shape[1]
    xg = x.reshape(b, _S, c)[:, _OFF:_OFF + _M, :].reshape(b, _G, _G, c)
    return xg[:, 1:57, 1:57, :]


def kernel(x, enc_w1, enc_b1, enc_w2, enc_b2, enc_w3, enc_b3, codebook,
           dec_w1, dec_b1, dec_w2, dec_b2, dec_w3, dec_b3):
    B = x.shape[0]

    xs = x.reshape(B, 56, 4, 56, 4).transpose(0, 1, 3, 2, 4).reshape(B, 56, 56, 16)
    xs = _to_grid(xs)

    h1 = _conv(xs, _remap(enc_w1, _MAP_ENC1), jnp.tile(enc_b1, 4), "relu")
    h2 = _conv(h1, _remap(enc_w2, _MAP_ENC2), enc_b2, "relu")
    z = _conv(h2, _remap(enc_w3, _MAP_ENC3), enc_b3)
    idx3, zq, sse = _vq(z, codebook)

    q_loss = sse / (B * 56 * 56 * 64)
    vq_loss = q_loss + 0.25 * q_loss

    idxg = idx3.reshape(B, _S)[:, _OFF:_OFF + _M].reshape(B, _G, _G)
    idx = idxg[:, 1:57, 1:57].reshape(B * 56 * 56)
    z_q_st = _from_grid(zq, B).transpose(0, 3, 1, 2)

    r1 = _conv(zq, _remap(dec_w1, _MAP_DEC1), dec_b1, "relu")
    r2 = _conv(r1, _remap(dec_w2, _MAP_DEC2), jnp.tile(dec_b2, 4), "relu")
    xr = _conv(r2, _remap(dec_w3, _MAP_DEC3), jnp.tile(dec_b3, 16), "sigmoid")

    xr = _from_grid(xr, B).reshape(B, 56, 56, 4, 4)
    x_recon = xr.transpose(0, 1, 3, 2, 4).reshape(B, 1, 224, 224)

    return x_recon, z_q_st, idx, vq_loss


# speedup vs baseline: 2.3092x; 2.3092x over previous
import functools

import jax
import jax.numpy as jnp
from jax import lax
from jax.experimental import pallas as pl
from jax.experimental.pallas import tpu as pltpu
from jax.experimental.pallas import tpu_sc as plsc

_G = 58
_M = _G * _G
_OFF = 64
_S = 3488
_V = 512


def _conv_body(in_ref, w_ref, b_ref, o_ref, *, act, cin):
    taps = [in_ref[pl.ds(_OFF + (dh - 1) * _G + (dw - 1), _M), pl.ds(0, cin)]
            for dh in range(3) for dw in range(3)]
    p = jnp.concatenate(taps, axis=1)
    acc = jnp.dot(p, w_ref[...], preferred_element_type=jnp.float32)
    acc = acc + b_ref[...]
    if act == "relu":
        acc = jnp.maximum(acc, 0.0)
    elif act == "sigmoid":
        acc = jax.nn.sigmoid(acc)
    q = jax.lax.broadcasted_iota(jnp.int32, (_M, 1), 0)
    r, c = q // _G, q % _G
    valid = (r >= 1) & (r <= 56) & (c >= 1) & (c <= 56)
    acc = jnp.where(valid, acc, 0.0)
    o_ref[pl.ds(0, _OFF), :] = jnp.zeros((_OFF, acc.shape[1]), jnp.float32)
    o_ref[pl.ds(_OFF, _M), :] = acc
    o_ref[pl.ds(_S - 64, 64), :] = jnp.zeros((64, acc.shape[1]), jnp.float32)


def _conv(x, w, b, act=None):
    bm, cin = x.shape
    nb = bm // _S
    k, n = w.shape
    return pl.pallas_call(
        functools.partial(_conv_body, act=act, cin=k // 9),
        grid=(nb,),
        in_specs=[
            pl.BlockSpec((_S, cin), lambda i: (i, 0)),
            pl.BlockSpec((k, n), lambda i: (0, 0)),
            pl.BlockSpec((1, n), lambda i: (0, 0)),
        ],
        out_specs=pl.BlockSpec((_S, n), lambda i: (i, 0)),
        out_shape=jax.ShapeDtypeStruct((bm, n), jnp.float32),
    )(x, w, b.reshape(1, n))


def _vq_body(z_ref, c_ref, idx_ref, zq_ref, loss_ref):
    z = z_ref[...]
    c = c_ref[...]
    s = jax.lax.dot_general(z, c, (((1,), (1,)), ((), ())),
                            preferred_element_type=jnp.float32)
    z2 = jnp.sum(z * z, axis=1, keepdims=True)
    c2 = jnp.sum(c * c, axis=1)[None, :]
    dists = z2 + c2 - 2.0 * s
    m = jnp.min(dists, axis=1, keepdims=True)
    iota = jax.lax.broadcasted_iota(jnp.int32, dists.shape, 1)
    idx = jnp.min(jnp.where(dists <= m, iota, _V), axis=1)

    q = jax.lax.broadcasted_iota(jnp.int32, (_S, 1), 0) - _OFF
    r, cc = q // _G, q % _G
    valid = (q >= 0) & (q < _M) & (r >= 1) & (r <= 56) & (cc >= 1) & (cc <= 56)
    idx_ref[0, 0, :] = jnp.where(valid[:, 0], idx, _V)
    oh = (iota == idx[:, None]).astype(jnp.float32)
    zq = jax.lax.dot_general(oh, c, (((1,), (0,)), ((), ())),
                             preferred_element_type=jnp.float32)
    zq_ref[...] = jnp.where(valid, zq, 0.0)
    part = jnp.sum(jnp.where(valid, m, 0.0)).reshape(1, 1)

    @pl.when(pl.program_id(0) == 0)
    def _():
        loss_ref[...] = jnp.zeros((1, 1), jnp.float32)

    loss_ref[...] += part


def _vq(z, codebook):
    bm, n = z.shape
    nb = bm // _S
    idx3, zq, loss = pl.pallas_call(
        _vq_body,
        grid=(nb,),
        in_specs=[
            pl.BlockSpec((_S, n), lambda i: (i, 0)),
            pl.BlockSpec((_V, n), lambda i: (0, 0)),
        ],
        out_specs=[
            pl.BlockSpec((1, 1, _S), lambda i: (i, 0, 0)),
            pl.BlockSpec((_S, n), lambda i: (i, 0)),
            pl.BlockSpec((1, 1), lambda i: (0, 0)),
        ],
        out_shape=[
            jax.ShapeDtypeStruct((nb, 1, _S), jnp.int32),
            jax.ShapeDtypeStruct((bm, n), jnp.float32),
            jax.ShapeDtypeStruct((1, 1), jnp.float32),
        ],
    )(z, codebook)
    return idx3, zq, loss[0, 0]


def _sc_gather(table, idx):
    nrows = idx.shape[0]
    d = table.shape[1]
    info = plsc.get_sparse_core_info()
    nw = info.num_cores * info.num_subcores
    b_per_w = nrows // nw
    nch = 4
    bc = b_per_w // nch
    mesh = plsc.VectorSubcoreMesh(core_axis_name="c", subcore_axis_name="s")

    @functools.partial(
        pl.kernel, mesh=mesh,
        out_type=jax.ShapeDtypeStruct((nrows, d), jnp.float32),
        scratch_types=[
            pltpu.VMEM((bc,), jnp.int32),
            pltpu.VMEM((bc, d), jnp.float32),
            pltpu.SemaphoreType.DMA,
        ],
    )
    def k(table_hbm, idx_hbm, out_hbm, idx_v, rows_v, sem):
        wid = lax.axis_index("s") * info.num_cores + lax.axis_index("c")
        base = wid * b_per_w
        for ch in range(nch):
            off = base + ch * bc
            pltpu.sync_copy(idx_hbm.at[pl.ds(off, bc)], idx_v)
            pltpu.async_copy(table_hbm.at[idx_v], rows_v, sem).wait()
            pltpu.sync_copy(rows_v, out_hbm.at[pl.ds(off, bc)])

    return k(table, idx)


def _fold4(v):
    return v // 4, v % 4


_PHASE_TAPS = {0: ((0, 0), (1, 2)), 1: ((1, 1), (2, 3))}


def _w_enc1(w):
    o = w.shape[0]
    ws = jnp.zeros((3, 3, 4, 4, 2, 2, o), jnp.float32)
    for fh in range(2):
        for th in range(4):
            dh, eh = _fold4(2 * fh + th - 1)
            for fw in range(2):
                for tw in range(4):
                    dw, ew = _fold4(2 * fw + tw - 1)
                    ws = ws.at[dh + 1, dw + 1, eh, ew, fh, fw, :].set(w[:, 0, th, tw])
    return ws.reshape(144, 4 * o)


def _w_enc2(w):
    o, i = w.shape[0], w.shape[1]
    ws = jnp.zeros((3, 3, 2, 2, i, o), jnp.float32)
    for th in range(4):
        dh, fh = (th - 1) // 2, (th - 1) % 2
        for tw in range(4):
            dw, fw = (tw - 1) // 2, (tw - 1) % 2
            ws = ws.at[dh + 1, dw + 1, fh, fw, :, :].set(w[:, :, th, tw].T)
    return ws.reshape(9 * 4 * i, o)


def _w_s1(w):
    return w.transpose(2, 3, 1, 0).reshape(-1, w.shape[0])


def _w_dec2(w):
    i, o = w.shape[0], w.shape[1]
    wf = jnp.flip(w, axis=(2, 3)).transpose(1, 0, 2, 3).transpose(2, 3, 1, 0)
    wc = jnp.zeros((3, 3, i, 2, 2, o), jnp.float32)
    for r in (0, 1):
        for s in (0, 1):
            for ph, dh in _PHASE_TAPS[r]:
                for pw, dw in _PHASE_TAPS[s]:
                    wc = wc.at[ph, pw, :, r, s, :].set(wf[dh, dw])
    return wc.reshape(9 * i, 4 * o)


def _w_dec3(w):
    i = w.shape[0]
    wf = jnp.flip(w, axis=(2, 3)).transpose(1, 0, 2, 3).transpose(2, 3, 1, 0)
    ws = jnp.zeros((3, 3, 2, 2, i, 4, 4), jnp.float32)
    for fh in range(2):
        for th in range(4):
            dh, eh = _fold4(2 * fh - th + 2)
            for fw in range(2):
                for tw in range(4):
                    dw, ew = _fold4(2 * fw - tw + 2)
                    ws = ws.at[1 - dh, 1 - dw, fh, fw, :, eh, ew].set(wf[th, tw, :, 0])
    return ws.reshape(9 * 4 * i, 16)


def _to_grid(x):
    b, _, _, c = x.shape
    xp = jnp.pad(x, ((0, 0), (1, 1), (1, 1), (0, 0))).reshape(b, _M, c)
    xp = jnp.pad(xp, ((0, 0), (_OFF, _S - _OFF - _M), (0, 0)))
    return xp.reshape(b * _S, c)


def _from_grid(x, b):
    c = x.shape[1]
    xg = x.reshape(b, _S, c)[:, _OFF:_OFF + _M, :].reshape(b, _G, _G, c)
    return xg[:, 1:57, 1:57, :]


def kernel(x, enc_w1, enc_b1, enc_w2, enc_b2, enc_w3, enc_b3, codebook,
           dec_w1, dec_b1, dec_w2, dec_b2, dec_w3, dec_b3):
    B = x.shape[0]

    xs = x.reshape(B, 56, 4, 56, 4).transpose(0, 1, 3, 2, 4).reshape(B, 56, 56, 16)
    xs = _to_grid(xs)

    h1 = _conv(xs, _w_enc1(enc_w1), jnp.tile(enc_b1, 4), "relu")
    h2 = _conv(h1, _w_enc2(enc_w2), enc_b2, "relu")
    z = _conv(h2, _w_s1(enc_w3), enc_b3)
    idx3, zq, sse = _vq(z, codebook)

    q_loss = sse / (B * 56 * 56 * 64)
    vq_loss = q_loss + 0.25 * q_loss

    idxg = idx3.reshape(B, _S)[:, _OFF:_OFF + _M].reshape(B, _G, _G)
    idx = idxg[:, 1:57, 1:57].reshape(B * 56 * 56)
    z_q_st = _from_grid(zq, B).transpose(0, 3, 1, 2)

    wd1 = jnp.flip(dec_w1, axis=(2, 3)).transpose(1, 0, 2, 3)
    r1 = _conv(zq, _w_s1(wd1), dec_b1, "relu")
    r2 = _conv(r1, _w_dec2(dec_w2), jnp.tile(dec_b2, 4), "relu")
    xr = _conv(r2, _w_dec3(dec_w3), jnp.tile(dec_b3, 16), "sigmoid")

    xr = _from_grid(xr, B).reshape(B, 56, 56, 4, 4)
    x_recon = xr.transpose(0, 1, 3, 2, 4).reshape(B, 1, 224, 224)

    return x_recon, z_q_st, idx, vq_loss


# speedup vs baseline: 2.3306x; 1.0093x over previous
import functools

import jax
import jax.numpy as jnp
from jax import lax
from jax.experimental import pallas as pl
from jax.experimental.pallas import tpu as pltpu
from jax.experimental.pallas import tpu_sc as plsc

_G = 58
_M = _G * _G
_OFF = 64
_S = 3488
_V = 512


def _conv_body(in_ref, w_ref, b_ref, o_ref, *, act, cin):
    taps = [in_ref[pl.ds(_OFF + (dh - 1) * _G + (dw - 1), _M), pl.ds(0, cin)]
            for dh in range(3) for dw in range(3)]
    p = jnp.concatenate(taps, axis=1)
    acc = jnp.dot(p, w_ref[...], preferred_element_type=jnp.float32)
    acc = acc + b_ref[...]
    if act == "relu":
        acc = jnp.maximum(acc, 0.0)
    elif act == "sigmoid":
        acc = jax.nn.sigmoid(acc)
    q = jax.lax.broadcasted_iota(jnp.int32, (_M, 1), 0)
    r, c = q // _G, q % _G
    valid = (r >= 1) & (r <= 56) & (c >= 1) & (c <= 56)
    acc = jnp.where(valid, acc, 0.0)
    o_ref[pl.ds(0, _OFF), :] = jnp.zeros((_OFF, acc.shape[1]), jnp.float32)
    o_ref[pl.ds(_OFF, _M), :] = acc
    o_ref[pl.ds(_S - 64, 64), :] = jnp.zeros((64, acc.shape[1]), jnp.float32)


def _conv(x, w, b, act=None):
    bm, cin = x.shape
    nb = bm // _S
    k, n = w.shape
    return pl.pallas_call(
        functools.partial(_conv_body, act=act, cin=k // 9),
        grid=(nb,),
        in_specs=[
            pl.BlockSpec((_S, cin), lambda i: (i, 0)),
            pl.BlockSpec((k, n), lambda i: (0, 0)),
            pl.BlockSpec((1, n), lambda i: (0, 0)),
        ],
        out_specs=pl.BlockSpec((_S, n), lambda i: (i, 0)),
        out_shape=jax.ShapeDtypeStruct((bm, n), jnp.float32),
    )(x, w, b.reshape(1, n))


def _vq_body(in_ref, w_ref, b_ref, c_ref, idx_ref, zq_ref, loss_ref):
    taps = [in_ref[pl.ds(_OFF + (dh - 1) * _G + (dw - 1), _M), :]
            for dh in range(3) for dw in range(3)]
    p = jnp.concatenate(taps, axis=1)
    acc = jnp.dot(p, w_ref[...], preferred_element_type=jnp.float32)
    acc = acc + b_ref[...]
    z = jnp.concatenate(
        [jnp.zeros((_OFF, acc.shape[1]), jnp.float32), acc,
         jnp.zeros((_S - _OFF - _M, acc.shape[1]), jnp.float32)], axis=0)
    c = c_ref[...]
    s = jax.lax.dot_general(z, c, (((1,), (1,)), ((), ())),
                            preferred_element_type=jnp.float32)
    z2 = jnp.sum(z * z, axis=1, keepdims=True)
    c2 = jnp.sum(c * c, axis=1)[None, :]
    dists = z2 + c2 - 2.0 * s
    m = jnp.min(dists, axis=1, keepdims=True)
    iota = jax.lax.broadcasted_iota(jnp.int32, dists.shape, 1)
    idx = jnp.min(jnp.where(dists <= m, iota, _V), axis=1)

    q = jax.lax.broadcasted_iota(jnp.int32, (_S, 1), 0) - _OFF
    r, cc = q // _G, q % _G
    valid = (q >= 0) & (q < _M) & (r >= 1) & (r <= 56) & (cc >= 1) & (cc <= 56)
    idx_ref[0, 0, :] = jnp.where(valid[:, 0], idx, _V)
    oh = (iota == idx[:, None]).astype(jnp.float32)
    zq = jax.lax.dot_general(oh, c, (((1,), (0,)), ((), ())),
                             preferred_element_type=jnp.float32)
    zq_ref[...] = jnp.where(valid, zq, 0.0)
    part = jnp.sum(jnp.where(valid, m, 0.0)).reshape(1, 1)

    @pl.when(pl.program_id(0) == 0)
    def _():
        loss_ref[...] = jnp.zeros((1, 1), jnp.float32)

    loss_ref[...] += part


def _vq(x, w, b, codebook):
    bm, cin = x.shape
    k, n = w.shape
    nb = bm // _S
    idx3, zq, loss = pl.pallas_call(
        _vq_body,
        grid=(nb,),
        in_specs=[
            pl.BlockSpec((_S, cin), lambda i: (i, 0)),
            pl.BlockSpec((k, n), lambda i: (0, 0)),
            pl.BlockSpec((1, n), lambda i: (0, 0)),
            pl.BlockSpec((_V, n), lambda i: (0, 0)),
        ],
        out_specs=[
            pl.BlockSpec((1, 1, _S), lambda i: (i, 0, 0)),
            pl.BlockSpec((_S, n), lambda i: (i, 0)),
            pl.BlockSpec((1, 1), lambda i: (0, 0)),
        ],
        out_shape=[
            jax.ShapeDtypeStruct((nb, 1, _S), jnp.int32),
            jax.ShapeDtypeStruct((bm, n), jnp.float32),
            jax.ShapeDtypeStruct((1, 1), jnp.float32),
        ],
    )(x, w, b.reshape(1, n), codebook)
    return idx3, zq, loss[0, 0]


def _sc_gather(table, idx):
    nrows = idx.shape[0]
    d = table.shape[1]
    info = plsc.get_sparse_core_info()
    nw = info.num_cores * info.num_subcores
    b_per_w = nrows // nw
    nch = 4
    bc = b_per_w // nch
    mesh = plsc.VectorSubcoreMesh(core_axis_name="c", subcore_axis_name="s")

    @functools.partial(
        pl.kernel, mesh=mesh,
        out_type=jax.ShapeDtypeStruct((nrows, d), jnp.float32),
        scratch_types=[
            pltpu.VMEM((bc,), jnp.int32),
            pltpu.VMEM((bc, d), jnp.float32),
            pltpu.SemaphoreType.DMA,
        ],
    )
    def k(table_hbm, idx_hbm, out_hbm, idx_v, rows_v, sem):
        wid = lax.axis_index("s") * info.num_cores + lax.axis_index("c")
        base = wid * b_per_w
        for ch in range(nch):
            off = base + ch * bc
            pltpu.sync_copy(idx_hbm.at[pl.ds(off, bc)], idx_v)
            pltpu.async_copy(table_hbm.at[idx_v], rows_v, sem).wait()
            pltpu.sync_copy(rows_v, out_hbm.at[pl.ds(off, bc)])

    return k(table, idx)


def _fold4(v):
    return v // 4, v % 4


_PHASE_TAPS = {0: ((0, 0), (1, 2)), 1: ((1, 1), (2, 3))}


def _w_enc1(w):
    o = w.shape[0]
    ws = jnp.zeros((3, 3, 4, 4, 2, 2, o), jnp.float32)
    for fh in range(2):
        for th in range(4):
            dh, eh = _fold4(2 * fh + th - 1)
            for fw in range(2):
                for tw in range(4):
                    dw, ew = _fold4(2 * fw + tw - 1)
                    ws = ws.at[dh + 1, dw + 1, eh, ew, fh, fw, :].set(w[:, 0, th, tw])
    return ws.reshape(144, 4 * o)


def _w_enc2(w):
    o, i = w.shape[0], w.shape[1]
    ws = jnp.zeros((3, 3, 2, 2, i, o), jnp.float32)
    for th in range(4):
        dh, fh = (th - 1) // 2, (th - 1) % 2
        for tw in range(4):
            dw, fw = (tw - 1) // 2, (tw - 1) % 2
            ws = ws.at[dh + 1, dw + 1, fh, fw, :, :].set(w[:, :, th, tw].T)
    return ws.reshape(9 * 4 * i, o)


def _w_s1(w):
    return w.transpose(2, 3, 1, 0).reshape(-1, w.shape[0])


def _w_dec2(w):
    i, o = w.shape[0], w.shape[1]
    wf = jnp.flip(w, axis=(2, 3)).transpose(1, 0, 2, 3).transpose(2, 3, 1, 0)
    wc = jnp.zeros((3, 3, i, 2, 2, o), jnp.float32)
    for r in (0, 1):
        for s in (0, 1):
            for ph, dh in _PHASE_TAPS[r]:
                for pw, dw in _PHASE_TAPS[s]:
                    wc = wc.at[ph, pw, :, r, s, :].set(wf[dh, dw])
    return wc.reshape(9 * i, 4 * o)


def _w_dec3(w):
    i = w.shape[0]
    wf = jnp.flip(w, axis=(2, 3)).transpose(1, 0, 2, 3).transpose(2, 3, 1, 0)
    ws = jnp.zeros((3, 3, 2, 2, i, 4, 4), jnp.float32)
    for fh in range(2):
        for th in range(4):
            dh, eh = _fold4(2 * fh - th + 2)
            for fw in range(2):
                for tw in range(4):
                    dw, ew = _fold4(2 * fw - tw + 2)
                    ws = ws.at[1 - dh, 1 - dw, fh, fw, :, eh, ew].set(wf[th, tw, :, 0])
    return ws.reshape(9 * 4 * i, 16)


def _to_grid(x):
    b, _, _, c = x.shape
    xp = jnp.pad(x, ((0, 0), (1, 1), (1, 1), (0, 0))).reshape(b, _M, c)
    xp = jnp.pad(xp, ((0, 0), (_OFF, _S - _OFF - _M), (0, 0)))
    return xp.reshape(b * _S, c)


def _from_grid(x, b):
    c = x.shape[1]
    xg = x.reshape(b, _S, c)[:, _OFF:_OFF + _M, :].reshape(b, _G, _G, c)
    return xg[:, 1:57, 1:57, :]


def kernel(x, enc_w1, enc_b1, enc_w2, enc_b2, enc_w3, enc_b3, codebook,
           dec_w1, dec_b1, dec_w2, dec_b2, dec_w3, dec_b3):
    B = x.shape[0]

    xs = x.reshape(B, 56, 4, 56, 4).transpose(0, 1, 3, 2, 4).reshape(B, 56, 56, 16)
    xs = _to_grid(xs)

    h1 = _conv(xs, _w_enc1(enc_w1), jnp.tile(enc_b1, 4), "relu")
    h2 = _conv(h1, _w_enc2(enc_w2), enc_b2, "relu")
    idx3, zq, sse = _vq(h2, _w_s1(enc_w3), enc_b3, codebook)

    q_loss = sse / (B * 56 * 56 * 64)
    vq_loss = q_loss + 0.25 * q_loss

    idxg = idx3.reshape(B, _S)[:, _OFF:_OFF + _M].reshape(B, _G, _G)
    idx = idxg[:, 1:57, 1:57].reshape(B * 56 * 56)
    z_q_st = _from_grid(zq, B).transpose(0, 3, 1, 2)

    wd1 = jnp.flip(dec_w1, axis=(2, 3)).transpose(1, 0, 2, 3)
    r1 = _conv(zq, _w_s1(wd1), dec_b1, "relu")
    r2 = _conv(r1, _w_dec2(dec_w2), jnp.tile(dec_b2, 4), "relu")
    xr = _conv(r2, _w_dec3(dec_w3), jnp.tile(dec_b3, 16), "sigmoid")

    xr = _from_grid(xr, B).reshape(B, 56, 56, 4, 4)
    x_recon = xr.transpose(0, 1, 3, 2, 4).reshape(B, 1, 224, 224)

    return x_recon, z_q_st, idx, vq_loss


# speedup vs baseline: 2.3314x; 1.0004x over previous
import functools

import jax
import jax.numpy as jnp
from jax.experimental import pallas as pl

_G = 58
_M = _G * _G
_OFF = 64
_S = 3488
_V = 512


def _conv_body(in_ref, w_ref, b_ref, o_ref, *, act, cin):
    taps = [in_ref[pl.ds(_OFF + (dh - 1) * _G + (dw - 1), _M), pl.ds(0, cin)]
            for dh in range(3) for dw in range(3)]
    p = jnp.concatenate(taps, axis=1)
    acc = jnp.dot(p, w_ref[...], preferred_element_type=jnp.float32)
    acc = acc + b_ref[...]
    if act == "relu":
        acc = jnp.maximum(acc, 0.0)
    elif act == "sigmoid":
        acc = jax.nn.sigmoid(acc)
    q = jax.lax.broadcasted_iota(jnp.int32, (_M, 1), 0)
    r, c = q // _G, q % _G
    valid = (r >= 1) & (r <= 56) & (c >= 1) & (c <= 56)
    acc = jnp.where(valid, acc, 0.0)
    o_ref[pl.ds(0, _OFF), :] = jnp.zeros((_OFF, acc.shape[1]), jnp.float32)
    o_ref[pl.ds(_OFF, _M), :] = acc
    o_ref[pl.ds(_S - 64, 64), :] = jnp.zeros((64, acc.shape[1]), jnp.float32)


def _conv(x, w, b, act=None):
    bm, cin = x.shape
    nb = bm // _S
    k, n = w.shape
    return pl.pallas_call(
        functools.partial(_conv_body, act=act, cin=k // 9),
        grid=(nb,),
        in_specs=[
            pl.BlockSpec((_S, cin), lambda i: (i, 0)),
            pl.BlockSpec((k, n), lambda i: (0, 0)),
            pl.BlockSpec((1, n), lambda i: (0, 0)),
        ],
        out_specs=pl.BlockSpec((_S, n), lambda i: (i, 0)),
        out_shape=jax.ShapeDtypeStruct((bm, n), jnp.float32),
    )(x, w, b.reshape(1, n))


def _vq_body(in_ref, w_ref, b_ref, c_ref, idx_ref, zq_ref, loss_ref):
    taps = [in_ref[pl.ds(_OFF + (dh - 1) * _G + (dw - 1), _M), :]
            for dh in range(3) for dw in range(3)]
    p = jnp.concatenate(taps, axis=1)
    acc = jnp.dot(p, w_ref[...], preferred_element_type=jnp.float32)
    acc = acc + b_ref[...]
    z = jnp.concatenate(
        [jnp.zeros((_OFF, acc.shape[1]), jnp.float32), acc,
         jnp.zeros((_S - _OFF - _M, acc.shape[1]), jnp.float32)], axis=0)
    c = c_ref[...]
    s = jax.lax.dot_general(z, c, (((1,), (1,)), ((), ())),
                            preferred_element_type=jnp.float32)
    z2 = jnp.sum(z * z, axis=1, keepdims=True)
    c2 = jnp.sum(c * c, axis=1)[None, :]
    dists = z2 + c2 - 2.0 * s
    m = jnp.min(dists, axis=1, keepdims=True)
    iota = jax.lax.broadcasted_iota(jnp.int32, dists.shape, 1)
    idx = jnp.min(jnp.where(dists <= m, iota, _V), axis=1)

    q = jax.lax.broadcasted_iota(jnp.int32, (_S, 1), 0) - _OFF
    r, cc = q // _G, q % _G
    valid = (q >= 0) & (q < _M) & (r >= 1) & (r <= 56) & (cc >= 1) & (cc <= 56)
    idx_ref[0, 0, :] = jnp.where(valid[:, 0], idx, _V)
    oh = (iota == idx[:, None]).astype(jnp.float32)
    zq = jax.lax.dot_general(oh, c, (((1,), (0,)), ((), ())),
                             preferred_element_type=jnp.float32)
    zq_ref[...] = jnp.where(valid, zq, 0.0)
    part = jnp.sum(jnp.where(valid, m, 0.0)).reshape(1, 1)

    @pl.when(pl.program_id(0) == 0)
    def _():
        loss_ref[...] = jnp.zeros((1, 1), jnp.float32)

    loss_ref[...] += part


def _vq(x, w, b, codebook):
    bm, cin = x.shape
    k, n = w.shape
    nb = bm // _S
    idx3, zq, loss = pl.pallas_call(
        _vq_body,
        grid=(nb,),
        in_specs=[
            pl.BlockSpec((_S, cin), lambda i: (i, 0)),
            pl.BlockSpec((k, n), lambda i: (0, 0)),
            pl.BlockSpec((1, n), lambda i: (0, 0)),
            pl.BlockSpec((_V, n), lambda i: (0, 0)),
        ],
        out_specs=[
            pl.BlockSpec((1, 1, _S), lambda i: (i, 0, 0)),
            pl.BlockSpec((_S, n), lambda i: (i, 0)),
            pl.BlockSpec((1, 1), lambda i: (0, 0)),
        ],
        out_shape=[
            jax.ShapeDtypeStruct((nb, 1, _S), jnp.int32),
            jax.ShapeDtypeStruct((bm, n), jnp.float32),
            jax.ShapeDtypeStruct((1, 1), jnp.float32),
        ],
    )(x, w, b.reshape(1, n), codebook)
    return idx3, zq, loss[0, 0]


def _fold4(v):
    return v // 4, v % 4


_PHASE_TAPS = {0: ((0, 0), (1, 2)), 1: ((1, 1), (2, 3))}


def _w_enc1(w):
    o = w.shape[0]
    ws = jnp.zeros((3, 3, 4, 4, 2, 2, o), jnp.float32)
    for fh in range(2):
        for th in range(4):
            dh, eh = _fold4(2 * fh + th - 1)
            for fw in range(2):
                for tw in range(4):
                    dw, ew = _fold4(2 * fw + tw - 1)
                    ws = ws.at[dh + 1, dw + 1, eh, ew, fh, fw, :].set(w[:, 0, th, tw])
    return ws.reshape(144, 4 * o)


def _w_enc2(w):
    o, i = w.shape[0], w.shape[1]
    ws = jnp.zeros((3, 3, 2, 2, i, o), jnp.float32)
    for th in range(4):
        dh, fh = (th - 1) // 2, (th - 1) % 2
        for tw in range(4):
            dw, fw = (tw - 1) // 2, (tw - 1) % 2
            ws = ws.at[dh + 1, dw + 1, fh, fw, :, :].set(w[:, :, th, tw].T)
    return ws.reshape(9 * 4 * i, o)


def _w_s1(w):
    return w.transpose(2, 3, 1, 0).reshape(-1, w.shape[0])


def _w_dec2(w):
    i, o = w.shape[0], w.shape[1]
    wf = jnp.flip(w, axis=(2, 3)).transpose(1, 0, 2, 3).transpose(2, 3, 1, 0)
    wc = jnp.zeros((3, 3, i, 2, 2, o), jnp.float32)
    for r in (0, 1):
        for s in (0, 1):
            for ph, dh in _PHASE_TAPS[r]:
                for pw, dw in _PHASE_TAPS[s]:
                    wc = wc.at[ph, pw, :, r, s, :].set(wf[dh, dw])
    return wc.reshape(9 * i, 4 * o)


def _w_dec3(w):
    i = w.shape[0]
    wf = jnp.flip(w, axis=(2, 3)).transpose(1, 0, 2, 3).transpose(2, 3, 1, 0)
    ws = jnp.zeros((3, 3, 2, 2, i, 4, 4), jnp.float32)
    for fh in range(2):
        for th in range(4):
            dh, eh = _fold4(2 * fh - th + 2)
            for fw in range(2):
                for tw in range(4):
                    dw, ew = _fold4(2 * fw - tw + 2)
                    ws = ws.at[1 - dh, 1 - dw, fh, fw, :, eh, ew].set(wf[th, tw, :, 0])
    return ws.reshape(9 * 4 * i, 16)


def _to_grid(x):
    b, _, _, c = x.shape
    xp = jnp.pad(x, ((0, 0), (1, 1), (1, 1), (0, 0))).reshape(b, _M, c)
    xp = jnp.pad(xp, ((0, 0), (_OFF, _S - _OFF - _M), (0, 0)))
    return xp.reshape(b * _S, c)


def _from_grid(x, b):
    c = x.shape[1]
    xg = x.reshape(b, _S, c)[:, _OFF:_OFF + _M, :].reshape(b, _G, _G, c)
    return xg[:, 1:57, 1:57, :]


def kernel(x, enc_w1, enc_b1, enc_w2, enc_b2, enc_w3, enc_b3, codebook,
           dec_w1, dec_b1, dec_w2, dec_b2, dec_w3, dec_b3):
    B = x.shape[0]

    xs = x.reshape(B, 56, 4, 56, 4).transpose(0, 1, 3, 2, 4).reshape(B, 56, 56, 16)
    xs = _to_grid(xs)

    h1 = _conv(xs, _w_enc1(enc_w1), jnp.tile(enc_b1, 4), "relu")
    h2 = _conv(h1, _w_enc2(enc_w2), enc_b2, "relu")
    idx3, zq, sse = _vq(h2, _w_s1(enc_w3), enc_b3, codebook)

    q_loss = sse / (B * 56 * 56 * 64)
    vq_loss = q_loss + 0.25 * q_loss

    idxg = idx3.reshape(B, _S)[:, _OFF:_OFF + _M].reshape(B, _G, _G)
    idx = idxg[:, 1:57, 1:57].reshape(B * 56 * 56)
    z_q_st = _from_grid(zq, B).transpose(0, 3, 1, 2)

    wd1 = jnp.flip(dec_w1, axis=(2, 3)).transpose(1, 0, 2, 3)
    r1 = _conv(zq, _w_s1(wd1), dec_b1, "relu")
    r2 = _conv(r1, _w_dec2(dec_w2), jnp.tile(dec_b2, 4), "relu")
    xr = _conv(r2, _w_dec3(dec_w3), jnp.tile(dec_b3, 16), "sigmoid")

    xr = _from_grid(xr, B).reshape(B, 56, 56, 4, 4)
    x_recon = xr.transpose(0, 1, 3, 2, 4).reshape(B, 1, 224, 224)

    return x_recon, z_q_st, idx, vq_loss
